# Initial kernel scaffold; baseline (speedup 1.0000x reference)
import functools

import jax
import jax.numpy as jnp
from jax import lax
from jax.experimental import pallas as pl
from jax.experimental.pallas import tpu as pltpu
from jax.experimental.pallas import tpu_sc as plsc

N = 10000
D = 128
R = 8
NB = 4
BS = D // NB
E2 = 2 * 320000

NC = 2
NS = 16
L = 16
NW = NC * NS
K = 128
CH = -(-E2 // (NW * K))
EP = NW * CH * K
ROWS_PER_TILE = N // NS


def _mm_body(x_ref, w_ref, o_ref):
    o_ref[...] = jnp.dot(x_ref[...], w_ref[...],
                         preferred_element_type=jnp.float32)


def _add_body(a_ref, b_ref, o_ref):
    o_ref[...] = a_ref[...] + b_ref[...]


def _sc_body(xr_hbm, gidx_hbm, tgt_hbm, w_hbm, zeros_hbm, out_hbm,
             gidx_v, tgt_v, w_v, rows_v, acc_sh, sem):
    cid = lax.axis_index("c")
    sid = lax.axis_index("s")
    wid = sid * NC + cid

    sl = pl.ds(sid * ROWS_PER_TILE, ROWS_PER_TILE)
    pltpu.sync_copy(zeros_hbm.at[sl], acc_sh.at[sl])

    pltpu.sync_copy(gidx_hbm.at[wid], gidx_v)
    pltpu.sync_copy(tgt_hbm.at[wid], tgt_v)
    pltpu.sync_copy(w_hbm.at[wid], w_v)

    plsc.subcore_barrier()

    def chunk_body(c, carry):
        pltpu.async_copy(xr_hbm.at[gidx_v.at[c]], rows_v, sem).wait()

        def row_body(k, carry2):
            wk = w_v[c, k]
            for f in range(D // L):
                fs = pl.ds(f * L, L)
                rows_v[k, fs] = rows_v[k, fs] * wk
            return carry2

        lax.fori_loop(0, K, row_body, 0)

        pltpu.sync_copy(rows_v, acc_sh.at[tgt_v.at[c]], add=True)
        return carry

    lax.fori_loop(0, CH, chunk_body, 0)

    plsc.subcore_barrier()

    pltpu.sync_copy(acc_sh.at[sl], out_hbm.at[cid].at[sl])


def kernel(x, source, target, edge_type, edge_weights, blocks):
    w8 = jnp.zeros((R, D, D), jnp.float32)
    for b in range(NB):
        s = b * BS
        w8 = w8.at[:, s:s + BS, s:s + BS].set(blocks[:R, b])
    wcat = jnp.transpose(w8, (1, 0, 2)).reshape(D, R * D)

    TN = 1000
    xr = pl.pallas_call(
        _mm_body,
        grid=(N // TN,),
        in_specs=[
            pl.BlockSpec((TN, D), lambda i: (i, 0)),
            pl.BlockSpec((D, R * D), lambda i: (0, 0)),
        ],
        out_specs=pl.BlockSpec((TN, R * D), lambda i: (i, 0)),
        out_shape=jax.ShapeDtypeStruct((N, R * D), jnp.float32),
    )(x, wcat)
    xr = xr.reshape(N * R, D)

    src_all = jnp.concatenate([source, target])
    tgt_all = jnp.concatenate([target, source])
    et_all = jnp.concatenate([edge_type, edge_type])
    w_all = jnp.concatenate([edge_weights, edge_weights])
    gidx = (src_all * R + et_all).astype(jnp.int32)

    pad = EP - E2
    gidx3 = jnp.pad(gidx, (0, pad)).reshape(NW, CH, K)
    tgt3 = jnp.pad(tgt_all.astype(jnp.int32), (0, pad)).reshape(NW, CH, K)
    w3 = jnp.pad(w_all, (0, pad)).reshape(NW, CH, K)
    zeros = jnp.zeros((N, D), jnp.float32)

    mesh = plsc.VectorSubcoreMesh(core_axis_name="c", subcore_axis_name="s")
    sc_kernel = functools.partial(
        pl.kernel,
        mesh=mesh,
        out_type=jax.ShapeDtypeStruct((NC, N, D), jnp.float32),
        scratch_types=[
            pltpu.VMEM((CH, K), jnp.int32),
            pltpu.VMEM((CH, K), jnp.int32),
            pltpu.VMEM((CH, K), jnp.float32),
            pltpu.VMEM((K, D), jnp.float32),
            pltpu.VMEM_SHARED((N, D), jnp.float32),
            pltpu.SemaphoreType.DMA,
        ],
    )(_sc_body)
    partials = sc_kernel(xr, gidx3, tgt3, w3, zeros)

    out = pl.pallas_call(
        _add_body,
        grid=(N // TN,),
        in_specs=[
            pl.BlockSpec((TN, D), lambda i: (i, 0)),
            pl.BlockSpec((TN, D), lambda i: (i, 0)),
        ],
        out_specs=pl.BlockSpec((TN, D), lambda i: (i, 0)),
        out_shape=jax.ShapeDtypeStruct((N, D), jnp.float32),
    )(partials[0], partials[1])
    return out


# speedup vs baseline: 482.0794x; 482.0794x over previous
import functools

import jax
import jax.numpy as jnp
from jax import lax
from jax.experimental import pallas as pl
from jax.experimental.pallas import tpu as pltpu
from jax.experimental.pallas import tpu_sc as plsc

N = 10000
D = 128
DH = D // 2
R = 8
NB = 4
BS = D // NB
E2 = 2 * 320000

NC = 2
NS = 16
L = 16
K = 128
PH = 2
CHP = -(-E2 // (NS * PH * K))
EP = NS * PH * CHP * K
NP = 10240
ROWS_PER_TILE = NP // NS


def _mm_body(x_ref, w_ref, o_ref):
    o_ref[...] = jnp.dot(x_ref[...], w_ref[...],
                         preferred_element_type=jnp.float32)


def _interleave_body(a_ref, b_ref, o_ref):
    o_ref[:, :DH] = a_ref[...]
    o_ref[:, DH:] = b_ref[...]


def _sc_body(xr_hbm, gidx_hbm, tgt_hbm, w_hbm, zeros_hbm, out_hbm,
             gidx_v, tgt_v, w_v, rows_v, acc_sh, sem):
    cid = lax.axis_index("c")
    sid = lax.axis_index("s")

    sl = pl.ds(sid * ROWS_PER_TILE, ROWS_PER_TILE)
    pltpu.sync_copy(zeros_hbm.at[sl], acc_sh.at[sl])
    plsc.subcore_barrier()

    for ph in range(PH):
        pltpu.sync_copy(gidx_hbm.at[sid].at[ph], gidx_v)
        pltpu.sync_copy(tgt_hbm.at[sid].at[ph], tgt_v)
        pltpu.sync_copy(w_hbm.at[sid].at[ph], w_v)

        def fix_body(c, carry):
            for f in range(K // L):
                fs = pl.ds(f * L, L)
                gidx_v[c, fs] = gidx_v[c, fs] + cid
            return carry

        lax.fori_loop(0, CHP, fix_body, 0)

        def chunk_body(c, carry):
            pltpu.async_copy(xr_hbm.at[gidx_v.at[c]], rows_v, sem).wait()

            def grp_body(g, carry2):
                wv = w_v[c, pl.ds(g * L, L)]
                for k in range(L):
                    wk = wv[k]
                    row = g * L + k
                    for f in range(DH // L):
                        fs = pl.ds(f * L, L)
                        rows_v[row, fs] = rows_v[row, fs] * wk
                return carry2

            lax.fori_loop(0, K // L, grp_body, 0)

            pltpu.sync_copy(rows_v, acc_sh.at[tgt_v.at[c]], add=True)
            return carry

        lax.fori_loop(0, CHP, chunk_body, 0)

    plsc.subcore_barrier()

    pltpu.sync_copy(acc_sh.at[sl], out_hbm.at[cid].at[sl])


def kernel(x, source, target, edge_type, edge_weights, blocks):
    w8 = jnp.zeros((R, D, D), jnp.float32)
    for b in range(NB):
        s = b * BS
        w8 = w8.at[:, s:s + BS, s:s + BS].set(blocks[:R, b])
    wcat = jnp.transpose(w8, (1, 0, 2)).reshape(D, R * D)

    TN = 1000
    xr = pl.pallas_call(
        _mm_body,
        grid=(N // TN,),
        in_specs=[
            pl.BlockSpec((TN, D), lambda i: (i, 0)),
            pl.BlockSpec((D, R * D), lambda i: (0, 0)),
        ],
        out_specs=pl.BlockSpec((TN, R * D), lambda i: (i, 0)),
        out_shape=jax.ShapeDtypeStruct((N, R * D), jnp.float32),
    )(x, wcat)
    xr = xr.reshape(N * R * 2, DH)

    src_all = jnp.concatenate([source, target])
    tgt_all = jnp.concatenate([target, source])
    et_all = jnp.concatenate([edge_type, edge_type])
    w_all = jnp.concatenate([edge_weights, edge_weights])
    gidx = ((src_all * R + et_all) * 2).astype(jnp.int32)

    pad = EP - E2
    shp = (NS, PH, CHP, K)
    gidx4 = jnp.pad(gidx, (0, pad)).reshape(shp)
    tgt4 = jnp.pad(tgt_all.astype(jnp.int32), (0, pad)).reshape(shp)
    w4 = jnp.pad(w_all, (0, pad)).reshape(shp)
    zeros = jnp.zeros((NP, DH), jnp.float32)

    mesh = plsc.VectorSubcoreMesh(core_axis_name="c", subcore_axis_name="s")
    sc_kernel = functools.partial(
        pl.kernel,
        mesh=mesh,
        compiler_params=pltpu.CompilerParams(use_tc_tiling_on_sc=False),
        out_type=jax.ShapeDtypeStruct((NC, NP, DH), jnp.float32),
        scratch_types=[
            pltpu.VMEM((CHP, K), jnp.int32),
            pltpu.VMEM((CHP, K), jnp.int32),
            pltpu.VMEM((CHP, K), jnp.float32),
            pltpu.VMEM((K, DH), jnp.float32),
            pltpu.VMEM_SHARED((NP, DH), jnp.float32),
            pltpu.SemaphoreType.DMA,
        ],
    )(_sc_body)
    partials = sc_kernel(xr, gidx4, tgt4, w4, zeros)

    out = pl.pallas_call(
        _interleave_body,
        grid=(N // TN,),
        in_specs=[
            pl.BlockSpec((TN, DH), lambda i: (i, 0)),
            pl.BlockSpec((TN, DH), lambda i: (i, 0)),
        ],
        out_specs=pl.BlockSpec((TN, D), lambda i: (i, 0)),
        out_shape=jax.ShapeDtypeStruct((N, D), jnp.float32),
    )(partials[0, :N], partials[1, :N])
    return out
